# baseline (speedup 1.0000x reference)
import functools

import jax
import jax.numpy as jnp
from jax import lax
from jax.experimental import pallas as pl
from jax.experimental.pallas import tpu as pltpu
from jax.experimental.pallas import tpu_sc as plsc

NC = 2
NS = 16
NW = NC * NS
ROWS = 2
NBUF = 8
NSPLIT = 2


@functools.lru_cache(maxsize=None)
def _make_gather(vocab, d, batch, seq):
    assert batch % (NW * ROWS) == 0
    b_per_w = batch // NW
    n_chunks = b_per_w // ROWS
    cids = ROWS * seq
    assert cids <= 128
    assert n_chunks % NBUF == 0 and n_chunks >= NBUF
    mesh = plsc.VectorSubcoreMesh(core_axis_name="c", subcore_axis_name="s")

    @functools.partial(
        pl.kernel,
        mesh=mesh,
        out_type=jax.ShapeDtypeStruct((batch, seq, d), jnp.float32),
        scratch_types=[
            pltpu.VMEM((n_chunks, cids), jnp.int32),
            pltpu.VMEM((NBUF, cids, d), jnp.float32),
        ]
        + [pltpu.SemaphoreType.DMA] * (2 * NBUF),
    )
    def gather_kernel(table_hbm, idx_hbm, out_hbm, idx_v, rows_v, *sems):
        gsem = sems[:NBUF]
        osem = sems[NBUF:]
        wid = lax.axis_index("s") * NC + lax.axis_index("c")
        base = wid * b_per_w
        pltpu.sync_copy(idx_hbm.at[wid], idx_v)

        def start_gather(g, b):
            pltpu.async_copy(table_hbm.at[idx_v.at[g]], rows_v.at[b], gsem[b])

        def wait_gather(g, b):
            pltpu.make_async_copy(
                table_hbm.at[idx_v.at[g]], rows_v.at[b], gsem[b]
            ).wait()

        def out_copies(g, b):
            for r in range(ROWS):
                yield (
                    rows_v.at[b, pl.ds(r * seq, seq)],
                    out_hbm.at[base + g * ROWS + r],
                    osem[b],
                )

        def start_out(g, b):
            for src, dst, sem in out_copies(g, b):
                pltpu.async_copy(src, dst, sem)

        def wait_out(g, b):
            for src, dst, sem in out_copies(g, b):
                pltpu.make_async_copy(src, dst, sem).wait()

        for c in range(NBUF - 1):
            start_gather(c, c)

        def outer(i, carry):
            go = i * NBUF
            for b in range(NBUF):
                g = go + b
                wait_gather(g, b)
                start_out(g, b)
                bn = (b + NBUF - 1) % NBUF
                gn = g + NBUF - 1

                @pl.when(g >= 1)
                def _():
                    wait_out(g - 1, bn)

                @pl.when(gn < n_chunks)
                def _():
                    start_gather(gn, bn)

            return carry

        lax.fori_loop(0, n_chunks // NBUF, outer, 0)
        wait_out(n_chunks - 1, (n_chunks - 1) % NBUF)

    return gather_kernel


def kernel(weights, token_ids):
    batch, seq = token_ids.shape
    vocab, d = weights.shape
    ids = token_ids.astype(jnp.int32)
    bsplit = batch // NSPLIT
    fn = _make_gather(vocab, d, bsplit, seq)
    outs = []
    for p in range(NSPLIT):
        part = ids[p * bsplit:(p + 1) * bsplit]
        ids3 = part.reshape(NW, bsplit // NW // ROWS, ROWS * seq)
        outs.append(fn(weights, ids3))
    return jnp.concatenate(outs, axis=0)


# speedup vs baseline: 1.5835x; 1.5835x over previous
import functools

import jax
import jax.numpy as jnp
from jax import lax
from jax.experimental import pallas as pl
from jax.experimental.pallas import tpu as pltpu
from jax.experimental.pallas import tpu_sc as plsc

NC = 2
NS = 16
NW = NC * NS
ROWS = 4
NBUF = 4


@functools.lru_cache(maxsize=None)
def _make_gather(vocab, d, batch, seq):
    assert batch % (NW * ROWS) == 0
    b_per_w = batch // NW
    n_chunks = b_per_w // ROWS
    assert seq <= 128
    assert n_chunks % NBUF == 0 and n_chunks >= NBUF
    mesh = plsc.VectorSubcoreMesh(core_axis_name="c", subcore_axis_name="s")

    @functools.partial(
        pl.kernel,
        mesh=mesh,
        out_type=jax.ShapeDtypeStruct((batch, seq, d), jnp.float32),
        scratch_types=[
            pltpu.VMEM((n_chunks, ROWS, seq), jnp.int32),
            pltpu.VMEM((NBUF, ROWS, seq, d), jnp.float32),
        ]
        + [pltpu.SemaphoreType.DMA] * (2 * NBUF),
    )
    def gather_kernel(table_hbm, idx_hbm, out_hbm, idx_v, rows_v, *sems):
        gsem = sems[:NBUF]
        osem = sems[NBUF:]
        wid = lax.axis_index("s") * NC + lax.axis_index("c")
        base = wid * b_per_w
        pltpu.sync_copy(idx_hbm.at[wid], idx_v)

        def gather_copies(g, b):
            for r in range(ROWS):
                yield (
                    table_hbm.at[idx_v.at[g, r]],
                    rows_v.at[b, r],
                    gsem[b],
                )

        def start_gather(g, b):
            for src, dst, sem in gather_copies(g, b):
                pltpu.async_copy(src, dst, sem)

        def wait_gather(g, b):
            for src, dst, sem in gather_copies(g, b):
                pltpu.make_async_copy(src, dst, sem).wait()

        def out_copy(g, b):
            return (
                rows_v.at[b],
                out_hbm.at[pl.ds(base + g * ROWS, ROWS)],
                osem[b],
            )

        def start_out(g, b):
            src, dst, sem = out_copy(g, b)
            pltpu.async_copy(src, dst, sem)

        def wait_out(g, b):
            src, dst, sem = out_copy(g, b)
            pltpu.make_async_copy(src, dst, sem).wait()

        for c in range(NBUF - 1):
            start_gather(c, c)

        def outer(i, carry):
            go = i * NBUF
            for b in range(NBUF):
                g = go + b
                wait_gather(g, b)
                start_out(g, b)
                bn = (b + NBUF - 1) % NBUF
                gn = g + NBUF - 1

                @pl.when(g >= 1)
                def _():
                    wait_out(g - 1, bn)

                @pl.when(gn < n_chunks)
                def _():
                    start_gather(gn, bn)

            return carry

        lax.fori_loop(0, n_chunks // NBUF, outer, 0)
        wait_out(n_chunks - 1, (n_chunks - 1) % NBUF)

    return gather_kernel


def kernel(weights, token_ids):
    batch, seq = token_ids.shape
    vocab, d = weights.shape
    ids = token_ids.astype(jnp.int32)
    ids4 = ids.reshape(NW, batch // NW // ROWS, ROWS, seq)
    return _make_gather(vocab, d, batch, seq)(weights, ids4)


# speedup vs baseline: 2.8565x; 1.8039x over previous
import functools

import jax
import jax.numpy as jnp
from jax import lax
from jax.experimental import pallas as pl
from jax.experimental.pallas import tpu as pltpu
from jax.experimental.pallas import tpu_sc as plsc

NC = 2
NS = 16
NW = NC * NS
NBUF = 5


@functools.lru_cache(maxsize=None)
def _make_gather(vocab, d, batch, seq):
    assert batch % NW == 0
    b_per_w = batch // NW
    assert b_per_w <= 128
    n_chunks = seq
    assert n_chunks % NBUF == 0 and n_chunks >= NBUF
    mesh = plsc.VectorSubcoreMesh(core_axis_name="c", subcore_axis_name="s")

    @functools.partial(
        pl.kernel,
        mesh=mesh,
        out_type=jax.ShapeDtypeStruct((seq, batch, d), jnp.float32),
        scratch_types=[
            pltpu.VMEM((n_chunks, b_per_w), jnp.int32),
            pltpu.VMEM((NBUF, b_per_w, d), jnp.float32),
        ]
        + [pltpu.SemaphoreType.DMA] * (2 * NBUF),
    )
    def gather_kernel(table_hbm, idx_hbm, out_hbm, idx_v, rows_v, *sems):
        gsem = sems[:NBUF]
        osem = sems[NBUF:]
        wid = lax.axis_index("s") * NC + lax.axis_index("c")
        base = wid * b_per_w
        pltpu.sync_copy(idx_hbm.at[:, pl.ds(base, b_per_w)], idx_v)

        def start_gather(g, b):
            pltpu.async_copy(table_hbm.at[idx_v.at[g]], rows_v.at[b], gsem[b])

        def wait_gather(g, b):
            pltpu.make_async_copy(
                table_hbm.at[idx_v.at[g]], rows_v.at[b], gsem[b]
            ).wait()

        def out_copy(g, b):
            return (
                rows_v.at[b],
                out_hbm.at[g, pl.ds(base, b_per_w)],
                osem[b],
            )

        def start_out(g, b):
            src, dst, sem = out_copy(g, b)
            pltpu.async_copy(src, dst, sem)

        def wait_out(g, b):
            src, dst, sem = out_copy(g, b)
            pltpu.make_async_copy(src, dst, sem).wait()

        for c in range(NBUF - 1):
            start_gather(c, c)

        def outer(i, carry):
            go = i * NBUF
            for b in range(NBUF):
                g = go + b
                wait_gather(g, b)
                start_out(g, b)
                bn = (b + NBUF - 1) % NBUF
                gn = g + NBUF - 1

                @pl.when(g >= 1)
                def _():
                    wait_out(g - 1, bn)

                @pl.when(gn < n_chunks)
                def _():
                    start_gather(gn, bn)

            return carry

        lax.fori_loop(0, n_chunks // NBUF, outer, 0)
        wait_out(n_chunks - 1, (n_chunks - 1) % NBUF)

    return gather_kernel


def kernel(weights, token_ids):
    batch, seq = token_ids.shape
    vocab, d = weights.shape
    ids_t = token_ids.astype(jnp.int32).T
    out_sbd = _make_gather(vocab, d, batch, seq)(weights, ids_t)
    return jnp.transpose(out_sbd, (1, 0, 2))
